# baseline (speedup 1.0000x reference)
import functools

import jax
import jax.numpy as jnp
from jax import lax
from jax.experimental import pallas as pl
from jax.experimental.pallas import tpu as pltpu
from jax.experimental.pallas import tpu_sc as plsc

_NC = 2
_NS = 16
_NW = _NC * _NS
_CHUNK = 128


@functools.lru_cache(maxsize=None)
def _make_mean_kernel(B, CTX, V, D):
    n_idx = B * CTX
    idx_per_w = n_idx // _NW
    b_per_w = B // _NW
    n_chunks = (idx_per_w + _CHUNK - 1) // _CHUNK
    assert idx_per_w % 16 == 0 and B % _NW == 0

    mesh = plsc.VectorSubcoreMesh(
        core_axis_name="c", subcore_axis_name="s",
        num_cores=_NC, num_subcores=_NS)

    @functools.partial(
        pl.kernel,
        mesh=mesh,
        compiler_params=pltpu.CompilerParams(needs_layout_passes=False),
        out_type=jax.ShapeDtypeStruct((B, D), jnp.float32),
        scratch_types=[
            pltpu.VMEM((idx_per_w,), jnp.int32),
            pltpu.VMEM((D, idx_per_w), jnp.int32),
            pltpu.VMEM((D, idx_per_w), jnp.float32),
            pltpu.VMEM((b_per_w, D), jnp.float32),
            pltpu.SemaphoreType.DMA,
        ],
    )
    def mean_kernel(tok_hbm, tablet_hbm, out_hbm, idx_v, idxd_v, rowst_v,
                    acc_v, sem):
        wid = lax.axis_index("s") * _NC + lax.axis_index("c")
        base = wid * idx_per_w
        pltpu.sync_copy(tok_hbm.at[pl.ds(base, idx_per_w)], idx_v)
        lanes = lax.iota(jnp.int32, 16)
        for d in range(D):
            for i in range(idx_per_w // 16):
                idxd_v[d, pl.ds(i * 16, 16)] = (
                    idx_v[pl.ds(i * 16, 16)] + jnp.int32(d * V))
        copies = []
        for d in range(D):
            for c in range(n_chunks):
                sz = min(_CHUNK, idx_per_w - c * _CHUNK)
                copies.append(pltpu.async_copy(
                    tablet_hbm.at[idxd_v.at[d, pl.ds(c * _CHUNK, sz)]],
                    rowst_v.at[d, pl.ds(c * _CHUNK, sz)],
                    sem))
        for cp in copies:
            cp.wait()
        inv = jnp.float32(1.0 / CTX)

        def row_body(r, carry):
            k0 = r * CTX
            acc = plsc.load_gather(rowst_v, [lanes, jnp.full((16,), k0, jnp.int32)])
            for j in range(1, CTX):
                acc = acc + plsc.load_gather(
                    rowst_v, [lanes, jnp.full((16,), k0 + j, jnp.int32)])
            acc_v[r] = acc * inv
            return carry

        lax.fori_loop(0, b_per_w, row_body, 0)
        pltpu.sync_copy(acc_v, out_hbm.at[pl.ds(wid * b_per_w, b_per_w)])

    return mean_kernel


@functools.lru_cache(maxsize=None)
def _make_proj_kernel(B, D, V, VT, BH, half):
    n_t = (V + VT - 1) // VT
    K = D + 1

    def proj_body(*refs):
        wt_ref, meant_ref, out_ref = refs[0], refs[1], refs[-1]
        out_ref[...] = lax.dot_general(
            wt_ref[...], meant_ref[...],
            (((0,), (0,)), ((), ())),
            preferred_element_type=jnp.float32,
        )

    in_specs = [
        pl.BlockSpec((K, VT), lambda i: (0, i)),
        pl.BlockSpec((K, BH), lambda i: (0, 0)),
    ]
    kwargs = {}
    if half == 1:
        in_specs.append(pl.BlockSpec(memory_space=pl.ANY))
        kwargs["input_output_aliases"] = {2: 0}
    return pl.pallas_call(
        proj_body,
        grid=(n_t,),
        in_specs=in_specs,
        out_specs=pl.BlockSpec((VT, BH), lambda i: (i, half)),
        out_shape=jax.ShapeDtypeStruct((V, B), jnp.float32),
        compiler_params=pltpu.CompilerParams(
            dimension_semantics=("arbitrary",)),
        **kwargs,
    )


def kernel(context_tokens, emb_table, W_out, b_out):
    B, CTX = context_tokens.shape
    V, D = emb_table.shape
    BH = B // 2
    tok = context_tokens.reshape(B * CTX).astype(jnp.int32)
    tablet = emb_table.T.reshape(V * D)
    mean_k = _make_mean_kernel(BH, CTX, V, D)
    mean0 = mean_k(tok[:BH * CTX], tablet)
    mean1 = mean_k(tok[BH * CTX:], tablet)
    wtb = jnp.concatenate([W_out.T, b_out[None, :]], axis=0)
    ones = jnp.ones((1, BH), jnp.float32)
    meant0 = jnp.concatenate([mean0.T, ones], axis=0)
    meant1 = jnp.concatenate([mean1.T, ones], axis=0)
    lt0 = _make_proj_kernel(B, D, V, 2048, BH, 0)(wtb, meant0)
    lt = _make_proj_kernel(B, D, V, 2048, BH, 1)(wtb, meant1, lt0)
    return lt.T


# speedup vs baseline: 1.1996x; 1.1996x over previous
import functools

import jax
import jax.numpy as jnp
from jax import lax
from jax.experimental import pallas as pl
from jax.experimental.pallas import tpu as pltpu
from jax.experimental.pallas import tpu_sc as plsc

_NC = 2
_NS = 16
_NW = _NC * _NS
_CHUNK = 128


@functools.lru_cache(maxsize=None)
def _make_mean_kernel(B, CTX, V, D):
    n_idx = B * CTX
    idx_per_w = n_idx // _NW
    b_per_w = B // _NW
    n_chunks = (idx_per_w + _CHUNK - 1) // _CHUNK
    assert idx_per_w % _CHUNK == 0 and B % _NW == 0

    mesh = plsc.VectorSubcoreMesh(
        core_axis_name="c", subcore_axis_name="s",
        num_cores=_NC, num_subcores=_NS)

    @functools.partial(
        pl.kernel,
        mesh=mesh,
        compiler_params=pltpu.CompilerParams(needs_layout_passes=False),
        out_type=jax.ShapeDtypeStruct((B, D), jnp.float32),
        scratch_types=[
            pltpu.VMEM((idx_per_w,), jnp.int32),
            pltpu.VMEM((D, idx_per_w), jnp.int32),
            pltpu.VMEM((D, idx_per_w), jnp.float32),
            pltpu.VMEM((b_per_w, D), jnp.float32),
            pltpu.SemaphoreType.DMA,
        ],
    )
    def mean_kernel(tok_hbm, tablet_hbm, out_hbm, idx_v, idxd_v, rowst_v,
                    acc_v, sem):
        wid = lax.axis_index("s") * _NC + lax.axis_index("c")
        base = wid * idx_per_w
        pltpu.sync_copy(tok_hbm.at[pl.ds(base, idx_per_w)], idx_v)
        lanes = lax.iota(jnp.int32, 16)
        for d in range(D):
            for i in range(idx_per_w // 16):
                idxd_v[d, pl.ds(i * 16, 16)] = (
                    idx_v[pl.ds(i * 16, 16)] + jnp.int32(d * V))
        copies = []
        for d in range(D):
            for c in range(n_chunks):
                copies.append(pltpu.async_copy(
                    tablet_hbm.at[idxd_v.at[d, pl.ds(c * _CHUNK, _CHUNK)]],
                    rowst_v.at[d, pl.ds(c * _CHUNK, _CHUNK)],
                    sem))
        for cp in copies:
            cp.wait()
        inv = jnp.float32(1.0 / CTX)

        def row_body(r, carry):
            k0 = r * CTX
            acc = plsc.load_gather(rowst_v, [lanes, jnp.full((16,), k0, jnp.int32)])
            for j in range(1, CTX):
                acc = acc + plsc.load_gather(
                    rowst_v, [lanes, jnp.full((16,), k0 + j, jnp.int32)])
            acc_v[r] = acc * inv
            return carry

        lax.fori_loop(0, b_per_w, row_body, 0)
        pltpu.sync_copy(acc_v, out_hbm.at[pl.ds(wid * b_per_w, b_per_w)])

    return mean_kernel


@functools.lru_cache(maxsize=None)
def _make_proj_kernel(B, D, V, VT):
    n_t = (V + VT - 1) // VT
    K = D + 1

    def proj_body(wt_ref, meant_ref, out_ref):
        out_ref[...] = lax.dot_general(
            wt_ref[...], meant_ref[...],
            (((0,), (0,)), ((), ())),
            preferred_element_type=jnp.float32,
        )

    return pl.pallas_call(
        proj_body,
        grid=(n_t,),
        in_specs=[
            pl.BlockSpec((K, VT), lambda i: (0, i)),
            pl.BlockSpec((K, B), lambda i: (0, 0)),
        ],
        out_specs=pl.BlockSpec((VT, B), lambda i: (i, 0)),
        out_shape=jax.ShapeDtypeStruct((V, B), jnp.float32),
        compiler_params=pltpu.CompilerParams(
            dimension_semantics=("arbitrary",)),
    )


def kernel(context_tokens, emb_table, W_out, b_out):
    B, CTX = context_tokens.shape
    V, D = emb_table.shape
    tok = context_tokens.reshape(B * CTX).astype(jnp.int32)
    tablet = emb_table.T.reshape(V * D)
    mean = _make_mean_kernel(B, CTX, V, D)(tok, tablet)
    wtb = jnp.concatenate([W_out.T, b_out[None, :]], axis=0)
    meant1 = jnp.concatenate(
        [mean.T, jnp.ones((1, B), jnp.float32)], axis=0)
    logits_t = _make_proj_kernel(B, D, V, 2048)(wtb, meant1)
    return logits_t.T
